# baseline (speedup 1.0000x reference)
import functools

import jax
import jax.numpy as jnp
from jax import lax
from jax.experimental import pallas as pl
from jax.experimental.pallas import tpu as pltpu
from jax.experimental.pallas import tpu_sc as plsc

_N = 10000
_E = 640000
_D = 128
_G = 16

_NC = 2
_NS = 16
_NW = _NC * _NS

_CHUNK = 128
_N_PAD = 10240
_ROWS_PER_TILE = _N_PAD // _NS
_IDXB = 40
_NBLK = 4
_N_CHUNKS = _IDXB * _NBLK
_E_PER_W = _N_CHUNKS * _CHUNK
_E_PAD = _E_PER_W * _NW
_DEG_W = 128


def _sc_mesh():
    return plsc.VectorSubcoreMesh(
        core_axis_name="c", subcore_axis_name="s",
        num_cores=_NC, num_subcores=_NS,
    )


def _deg_body(dst_hbm, ones_hbm, out_hbm, ones_v, zeros_v, didx_v, acc_s):
    cid = lax.axis_index("c")
    sid = lax.axis_index("s")
    wid = sid * _NC + cid

    pltpu.sync_copy(ones_hbm.at[0], ones_v)
    pltpu.sync_copy(ones_hbm.at[1], zeros_v)

    def zero_blk(i, _):
        pltpu.sync_copy(
            zeros_v,
            acc_s.at[pl.ds(sid * _ROWS_PER_TILE + i * _CHUNK, _CHUNK)],
        )
        return 0

    lax.fori_loop(0, _ROWS_PER_TILE // _CHUNK, zero_blk, 0)
    plsc.subcore_barrier()

    def blk(t, _):
        pltpu.sync_copy(dst_hbm.at[wid, pl.ds(t * _IDXB, _IDXB)], didx_v)

        def body(i, _):
            pltpu.sync_copy(ones_v, acc_s.at[didx_v.at[i]], add=True)
            return 0

        lax.fori_loop(0, _IDXB, body, 0)
        return 0

    lax.fori_loop(0, _NBLK, blk, 0)
    plsc.subcore_barrier()

    pltpu.sync_copy(
        acc_s.at[pl.ds(sid * _ROWS_PER_TILE, _ROWS_PER_TILE)],
        out_hbm.at[cid, pl.ds(sid * _ROWS_PER_TILE, _ROWS_PER_TILE)],
    )


@functools.cache
def _deg_kernel():
    return pl.kernel(
        _deg_body,
        out_type=jax.ShapeDtypeStruct((_NC, _N_PAD, _DEG_W), jnp.float32),
        mesh=_sc_mesh(),
        scratch_types=[
            pltpu.VMEM((_CHUNK, _DEG_W), jnp.float32),
            pltpu.VMEM((_CHUNK, _DEG_W), jnp.float32),
            pltpu.VMEM((_IDXB, _CHUNK), jnp.int32),
            pltpu.VMEM_SHARED((_N_PAD, _DEG_W), jnp.float32),
        ],
    )


def _agg_body(y_hbm, src_hbm, dst_hbm, out_hbm, sidx_v, didx_v, rows0_v, rows1_v,
              acc_s, sem0, sem1):
    cid = lax.axis_index("c")
    sid = lax.axis_index("s")
    wid = sid * _NC + cid

    def init_blk(i, _):
        r0 = sid * _ROWS_PER_TILE + i * _CHUNK
        pltpu.sync_copy(y_hbm.at[pl.ds(r0, _CHUNK)], acc_s.at[pl.ds(r0, _CHUNK)])
        return 0

    lax.fori_loop(0, _ROWS_PER_TILE // _CHUNK, init_blk, 0)
    plsc.subcore_barrier()

    def blk(t, _):
        pltpu.sync_copy(src_hbm.at[wid, pl.ds(t * _IDXB, _IDXB)], sidx_v)
        pltpu.sync_copy(dst_hbm.at[wid, pl.ds(t * _IDXB, _IDXB)], didx_v)

        def body(g, _):
            pltpu.async_copy(y_hbm.at[sidx_v.at[g]], rows0_v, sem0).wait()
            pltpu.sync_copy(rows0_v, acc_s.at[didx_v.at[g]], add=True)
            return 0

        lax.fori_loop(0, _IDXB, body, 0)
        return 0

    lax.fori_loop(0, _NBLK, blk, 0)
    plsc.subcore_barrier()

    def out_blk(i, _):
        r0 = sid * _ROWS_PER_TILE + i * _CHUNK
        pltpu.sync_copy(acc_s.at[pl.ds(r0, _CHUNK)], out_hbm.at[cid, pl.ds(r0, _CHUNK)])
        return 0

    lax.fori_loop(0, _ROWS_PER_TILE // _CHUNK, out_blk, 0)


@functools.cache
def _agg_kernel():
    return pl.kernel(
        _agg_body,
        out_type=jax.ShapeDtypeStruct((_NC, _N_PAD, _D), jnp.float32),
        mesh=_sc_mesh(),
        scratch_types=[
            pltpu.VMEM((_IDXB, _CHUNK), jnp.int32),
            pltpu.VMEM((_IDXB, _CHUNK), jnp.int32),
            pltpu.VMEM((_CHUNK, _D), jnp.float32),
            pltpu.VMEM((_CHUNK, _D), jnp.float32),
            pltpu.VMEM_SHARED((_N_PAD, _D), jnp.float32),
            pltpu.SemaphoreType.DMA,
            pltpu.SemaphoreType.DMA,
        ],
    )


_BM = 512
_N_BLOCKS = _N_PAD // _BM


def _dis_block(degp):
    deg = degp[0, :, 0:1] + degp[1, :, 0:1] + 1.0
    return lax.rsqrt(deg)


def _b_kernel(x_ref, degp_ref, w1_ref, y1_ref):
    dis = _dis_block(degp_ref[...])
    y1_ref[...] = (x_ref[...] @ w1_ref[...]) * dis


def _d_kernel(aggp_ref, y1_ref, degp_ref, w2_ref, b1_ref, y2_ref):
    dis = _dis_block(degp_ref[...])
    agg = aggp_ref[0] + aggp_ref[1] - y1_ref[...]
    h1 = jnp.maximum(agg * dis + b1_ref[...], 0.0)
    y2_ref[...] = (h1 @ w2_ref[...]) * dis


def _f_kernel(aggp_ref, y2_ref, degp_ref, batch_ref, b2_ref, wf_ref, bf_ref,
              wo_ref, bo_ref, out_ref, sum_s, cnt_s):
    i = pl.program_id(0)
    dis = _dis_block(degp_ref[...])
    agg = aggp_ref[0] + aggp_ref[1] - y2_ref[...]
    h2 = agg * dis + b2_ref[...]

    b = batch_ref[0, 0, :].reshape(_BM, 1)
    gid = lax.broadcasted_iota(jnp.int32, (_BM, _G), 1)
    p = jnp.where(b == gid, 1.0, 0.0)

    contract = (((0,), (0,)), ((), ()))
    psum = lax.dot_general(p, h2, contract,
                           precision=lax.Precision.HIGHEST)
    pcnt = lax.dot_general(p, jnp.ones((_BM, _D), jnp.float32), contract)

    @pl.when(i == 0)
    def _():
        sum_s[...] = jnp.zeros_like(sum_s)
        cnt_s[...] = jnp.zeros_like(cnt_s)

    sum_s[...] += psum
    cnt_s[...] += pcnt

    @pl.when(i == _N_BLOCKS - 1)
    def _():
        pooled = sum_s[...] / jnp.maximum(cnt_s[...], 1.0)
        hf = jnp.maximum(pooled @ wf_ref[...] + bf_ref[...], 0.0)
        out_ref[...] = hf @ wo_ref[...] + bo_ref[...]


def _row_spec(width):
    return pl.BlockSpec((_BM, width), lambda i: (i, 0))


def _part_spec(width):
    return pl.BlockSpec((_NC, _BM, width), lambda i: (0, i, 0))


def _full_spec(shape):
    return pl.BlockSpec(shape, lambda i: tuple(0 for _ in shape))


_b_call = pl.pallas_call(
    _b_kernel,
    grid=(_N_BLOCKS,),
    in_specs=[_row_spec(_D), _part_spec(_DEG_W), _full_spec((_D, _D))],
    out_specs=_row_spec(_D),
    out_shape=jax.ShapeDtypeStruct((_N_PAD, _D), jnp.float32),
)

_d_call = pl.pallas_call(
    _d_kernel,
    grid=(_N_BLOCKS,),
    in_specs=[_part_spec(_D), _row_spec(_D), _part_spec(_DEG_W),
              _full_spec((_D, _D)), _full_spec((1, _D))],
    out_specs=_row_spec(_D),
    out_shape=jax.ShapeDtypeStruct((_N_PAD, _D), jnp.float32),
)

_f_call = pl.pallas_call(
    _f_kernel,
    grid=(_N_BLOCKS,),
    in_specs=[_part_spec(_D), _row_spec(_D), _part_spec(_DEG_W),
              pl.BlockSpec((1, 1, _BM), lambda i: (i, 0, 0)),
              _full_spec((1, _D)), _full_spec((_D, 64)), _full_spec((1, 64)),
              _full_spec((64, _D)), _full_spec((1, _D))],
    out_specs=_full_spec((_G, _D)),
    out_shape=jax.ShapeDtypeStruct((_G, _D), jnp.float32),
    scratch_shapes=[pltpu.VMEM((_G, _D), jnp.float32),
                    pltpu.VMEM((_G, _D), jnp.float32)],
)


@jax.jit
def kernel(x, edge_index, batch, W1, b1, W2, b2, Wf, bf, Wo, bo):
    src = jnp.pad(edge_index[0], (0, _E_PAD - _E),
                  constant_values=_N).reshape(_NW, _N_CHUNKS, _CHUNK)
    dst = jnp.pad(edge_index[1], (0, _E_PAD - _E),
                  constant_values=_N).reshape(_NW, _N_CHUNKS, _CHUNK)
    x_pad = jnp.pad(x, ((0, _N_PAD - _N), (0, 0)))
    batch_pad = jnp.pad(batch, (0, _N_PAD - _N), constant_values=_G)
    batch_pad = batch_pad.reshape(_N_BLOCKS, 1, _BM)

    onehot_rows = jnp.zeros((2, _CHUNK, _DEG_W), jnp.float32).at[0, :, 0].set(1.0)
    degp = _deg_kernel()(dst, onehot_rows)
    y1 = _b_call(x_pad, degp, W1)
    agg1 = _agg_kernel()(y1, src, dst)
    y2 = _d_call(agg1, y1, degp, W2, b1.reshape(1, _D))
    agg2 = _agg_kernel()(y2, src, dst)

    wo_pad = jnp.pad(Wo, ((0, 0), (0, _D - 1)))
    bo_pad = jnp.pad(bo, (0, _D - 1)).reshape(1, _D)
    out = _f_call(agg2, y2, degp, batch_pad, b2.reshape(1, _D),
                  Wf, bf.reshape(1, 64), wo_pad, bo_pad)
    return out[:, 0:1]


# speedup vs baseline: 1.1101x; 1.1101x over previous
import functools

import jax
import jax.numpy as jnp
from jax import lax
from jax.experimental import pallas as pl
from jax.experimental.pallas import tpu as pltpu
from jax.experimental.pallas import tpu_sc as plsc

_N = 10000
_E = 640000
_D = 128
_G = 16

_NC = 2
_NS = 16
_NW = _NC * _NS

_CHUNK = 128
_N_PAD = 10240
_ROWS_PER_TILE = _N_PAD // _NS
_N_CHUNKS = 158
_E_PER_W = _N_CHUNKS * _CHUNK
_E_PAD = _E_PER_W * _NW
_DEG_W = 128


def _sc_mesh():
    return plsc.VectorSubcoreMesh(
        core_axis_name="c", subcore_axis_name="s",
        num_cores=_NC, num_subcores=_NS,
    )


def _deg_body(eidx_hbm, ones_hbm, out_hbm, ones_v, zeros_v, idx_v, acc_s):
    cid = lax.axis_index("c")
    sid = lax.axis_index("s")
    wid = sid * _NC + cid

    pltpu.sync_copy(ones_hbm.at[0], ones_v)
    pltpu.sync_copy(ones_hbm.at[1], zeros_v)

    def zero_blk(i, _):
        pltpu.sync_copy(
            zeros_v,
            acc_s.at[pl.ds(sid * _ROWS_PER_TILE + i * _CHUNK, _CHUNK)],
        )
        return 0

    lax.fori_loop(0, _ROWS_PER_TILE // _CHUNK, zero_blk, 0)
    plsc.subcore_barrier()

    base = wid * _N_CHUNKS

    def body(i, _):
        pltpu.sync_copy(eidx_hbm.at[base + i], idx_v)
        pltpu.sync_copy(ones_v, acc_s.at[idx_v.at[1]], add=True)
        return 0

    lax.fori_loop(0, _N_CHUNKS, body, 0)
    plsc.subcore_barrier()

    pltpu.sync_copy(
        acc_s.at[pl.ds(sid * _ROWS_PER_TILE, _ROWS_PER_TILE)],
        out_hbm.at[cid, pl.ds(sid * _ROWS_PER_TILE, _ROWS_PER_TILE)],
    )


@functools.cache
def _deg_kernel():
    return pl.kernel(
        _deg_body,
        out_type=jax.ShapeDtypeStruct((_NC, _N_PAD, _DEG_W), jnp.float32),
        mesh=_sc_mesh(),
        scratch_types=[
            pltpu.VMEM((_CHUNK, _DEG_W), jnp.float32),
            pltpu.VMEM((_CHUNK, _DEG_W), jnp.float32),
            pltpu.VMEM((2, _CHUNK), jnp.int32),
            pltpu.VMEM_SHARED((_N_PAD, _DEG_W), jnp.float32),
        ],
    )


def _agg_body(y_hbm, eidx_hbm, out_hbm, idx_v, rows0_v, acc_s, sem0):
    cid = lax.axis_index("c")
    sid = lax.axis_index("s")
    wid = sid * _NC + cid

    def init_blk(i, _):
        r0 = sid * _ROWS_PER_TILE + i * _CHUNK
        pltpu.sync_copy(y_hbm.at[pl.ds(r0, _CHUNK)], acc_s.at[pl.ds(r0, _CHUNK)])
        return 0

    lax.fori_loop(0, _ROWS_PER_TILE // _CHUNK, init_blk, 0)
    plsc.subcore_barrier()

    base = wid * _N_CHUNKS

    def body(i, _):
        pltpu.sync_copy(eidx_hbm.at[base + i], idx_v)
        pltpu.async_copy(y_hbm.at[idx_v.at[0]], rows0_v, sem0).wait()
        pltpu.sync_copy(rows0_v, acc_s.at[idx_v.at[1]], add=True)
        return 0

    lax.fori_loop(0, _N_CHUNKS, body, 0)
    plsc.subcore_barrier()

    def out_blk(i, _):
        r0 = sid * _ROWS_PER_TILE + i * _CHUNK
        pltpu.sync_copy(acc_s.at[pl.ds(r0, _CHUNK)], out_hbm.at[cid, pl.ds(r0, _CHUNK)])
        return 0

    lax.fori_loop(0, _ROWS_PER_TILE // _CHUNK, out_blk, 0)


@functools.cache
def _agg_kernel():
    return pl.kernel(
        _agg_body,
        out_type=jax.ShapeDtypeStruct((_NC, _N_PAD, _D), jnp.float32),
        mesh=_sc_mesh(),
        scratch_types=[
            pltpu.VMEM((2, _CHUNK), jnp.int32),
            pltpu.VMEM((_CHUNK, _D), jnp.float32),
            pltpu.VMEM_SHARED((_N_PAD, _D), jnp.float32),
            pltpu.SemaphoreType.DMA,
        ],
    )


_BM = 512
_N_BLOCKS = _N_PAD // _BM


def _dis_block(degp):
    deg = degp[0, :, 0:1] + degp[1, :, 0:1] + 1.0
    return lax.rsqrt(deg)


def _b_kernel(x_ref, degp_ref, w1_ref, y1_ref):
    dis = _dis_block(degp_ref[...])
    y1_ref[...] = (x_ref[...] @ w1_ref[...]) * dis


def _d_kernel(aggp_ref, y1_ref, degp_ref, w2_ref, b1_ref, y2_ref):
    dis = _dis_block(degp_ref[...])
    agg = aggp_ref[0] + aggp_ref[1] - y1_ref[...]
    h1 = jnp.maximum(agg * dis + b1_ref[...], 0.0)
    y2_ref[...] = (h1 @ w2_ref[...]) * dis


def _f_kernel(aggp_ref, y2_ref, degp_ref, batch_ref, b2_ref, wf_ref, bf_ref,
              wo_ref, bo_ref, out_ref, sum_s, cnt_s):
    i = pl.program_id(0)
    dis = _dis_block(degp_ref[...])
    agg = aggp_ref[0] + aggp_ref[1] - y2_ref[...]
    h2 = agg * dis + b2_ref[...]

    b = batch_ref[0, 0, :].reshape(_BM, 1)
    gid = lax.broadcasted_iota(jnp.int32, (_BM, _G), 1)
    p = jnp.where(b == gid, 1.0, 0.0)

    contract = (((0,), (0,)), ((), ()))
    psum = lax.dot_general(p, h2, contract,
                           precision=lax.Precision.HIGHEST)
    pcnt = lax.dot_general(p, jnp.ones((_BM, _D), jnp.float32), contract)

    @pl.when(i == 0)
    def _():
        sum_s[...] = jnp.zeros_like(sum_s)
        cnt_s[...] = jnp.zeros_like(cnt_s)

    sum_s[...] += psum
    cnt_s[...] += pcnt

    @pl.when(i == _N_BLOCKS - 1)
    def _():
        pooled = sum_s[...] / jnp.maximum(cnt_s[...], 1.0)
        hf = jnp.maximum(pooled @ wf_ref[...] + bf_ref[...], 0.0)
        out_ref[...] = hf @ wo_ref[...] + bo_ref[...]


def _row_spec(width):
    return pl.BlockSpec((_BM, width), lambda i: (i, 0))


def _part_spec(width):
    return pl.BlockSpec((_NC, _BM, width), lambda i: (0, i, 0))


def _full_spec(shape):
    return pl.BlockSpec(shape, lambda i: tuple(0 for _ in shape))


_b_call = pl.pallas_call(
    _b_kernel,
    grid=(_N_BLOCKS,),
    in_specs=[_row_spec(_D), _part_spec(_DEG_W), _full_spec((_D, _D))],
    out_specs=_row_spec(_D),
    out_shape=jax.ShapeDtypeStruct((_N_PAD, _D), jnp.float32),
)

_d_call = pl.pallas_call(
    _d_kernel,
    grid=(_N_BLOCKS,),
    in_specs=[_part_spec(_D), _row_spec(_D), _part_spec(_DEG_W),
              _full_spec((_D, _D)), _full_spec((1, _D))],
    out_specs=_row_spec(_D),
    out_shape=jax.ShapeDtypeStruct((_N_PAD, _D), jnp.float32),
)

_f_call = pl.pallas_call(
    _f_kernel,
    grid=(_N_BLOCKS,),
    in_specs=[_part_spec(_D), _row_spec(_D), _part_spec(_DEG_W),
              pl.BlockSpec((1, 1, _BM), lambda i: (i, 0, 0)),
              _full_spec((1, _D)), _full_spec((_D, 64)), _full_spec((1, 64)),
              _full_spec((64, _D)), _full_spec((1, _D))],
    out_specs=_full_spec((_G, _D)),
    out_shape=jax.ShapeDtypeStruct((_G, _D), jnp.float32),
    scratch_shapes=[pltpu.VMEM((_G, _D), jnp.float32),
                    pltpu.VMEM((_G, _D), jnp.float32)],
)


@jax.jit
def kernel(x, edge_index, batch, W1, b1, W2, b2, Wf, bf, Wo, bo):
    eidx = jnp.pad(edge_index, ((0, 0), (0, _E_PAD - _E)), constant_values=_N)
    eidx = eidx.reshape(2, _NW * _N_CHUNKS, _CHUNK).transpose(1, 0, 2)
    x_pad = jnp.pad(x, ((0, _N_PAD - _N), (0, 0)))
    batch_pad = jnp.pad(batch, (0, _N_PAD - _N), constant_values=_G)
    batch_pad = batch_pad.reshape(_N_BLOCKS, 1, _BM)

    onehot_rows = jnp.zeros((2, _CHUNK, _DEG_W), jnp.float32).at[0, :, 0].set(1.0)
    degp = _deg_kernel()(eidx, onehot_rows)
    y1 = _b_call(x_pad, degp, W1)
    agg1 = _agg_kernel()(y1, eidx)
    y2 = _d_call(agg1, y1, degp, W2, b1.reshape(1, _D))
    agg2 = _agg_kernel()(y2, eidx)

    wo_pad = jnp.pad(Wo, ((0, 0), (0, _D - 1)))
    bo_pad = jnp.pad(bo, (0, _D - 1)).reshape(1, _D)
    out = _f_call(agg2, y2, degp, batch_pad, b2.reshape(1, _D),
                  Wf, bf.reshape(1, 64), wo_pad, bo_pad)
    return out[:, 0:1]


# speedup vs baseline: 1.1773x; 1.0606x over previous
import functools

import jax
import jax.numpy as jnp
from jax import lax
from jax.experimental import pallas as pl
from jax.experimental.pallas import tpu as pltpu
from jax.experimental.pallas import tpu_sc as plsc

_N = 10000
_E = 640000
_D = 128
_G = 16

_NC = 2
_NS = 16
_NW = _NC * _NS

_CHUNK = 128
_N_PAD = 10240
_ROWS_PER_TILE = _N_PAD // _NS
_N_CHUNKS = 158
_E_PER_W = _N_CHUNKS * _CHUNK
_E_PAD = _E_PER_W * _NW
_DEG_W = 128


def _sc_mesh():
    return plsc.VectorSubcoreMesh(
        core_axis_name="c", subcore_axis_name="s",
        num_cores=_NC, num_subcores=_NS,
    )


def _deg_body(dst_hbm, ones_hbm, out_hbm, ones_v, zeros_v, idx_v, acc_s):
    cid = lax.axis_index("c")
    sid = lax.axis_index("s")
    wid = sid * _NC + cid

    pltpu.sync_copy(ones_hbm.at[0], ones_v)
    pltpu.sync_copy(ones_hbm.at[1], zeros_v)

    def zero_blk(i, _):
        pltpu.sync_copy(
            zeros_v,
            acc_s.at[pl.ds(sid * _ROWS_PER_TILE + i * _CHUNK, _CHUNK)],
        )
        return 0

    lax.fori_loop(0, _ROWS_PER_TILE // _CHUNK, zero_blk, 0)
    plsc.subcore_barrier()

    base = wid * _E_PER_W

    def body(i, _):
        pltpu.sync_copy(dst_hbm.at[pl.ds(base + i * _CHUNK, _CHUNK)], idx_v)
        pltpu.sync_copy(ones_v, acc_s.at[idx_v], add=True)
        return 0

    lax.fori_loop(0, _N_CHUNKS, body, 0)
    plsc.subcore_barrier()

    pltpu.sync_copy(
        acc_s.at[pl.ds(sid * _ROWS_PER_TILE, _ROWS_PER_TILE)],
        out_hbm.at[cid, pl.ds(sid * _ROWS_PER_TILE, _ROWS_PER_TILE)],
    )


@functools.cache
def _deg_kernel():
    return pl.kernel(
        _deg_body,
        out_type=jax.ShapeDtypeStruct((_NC, _N_PAD, _DEG_W), jnp.float32),
        mesh=_sc_mesh(),
        scratch_types=[
            pltpu.VMEM((_CHUNK, _DEG_W), jnp.float32),
            pltpu.VMEM((_CHUNK, _DEG_W), jnp.float32),
            pltpu.VMEM((_CHUNK,), jnp.int32),
            pltpu.VMEM_SHARED((_N_PAD, _DEG_W), jnp.float32),
        ],
    )


def _agg_body(y_hbm, src_hbm, dst_hbm, out_hbm, sidx_v, didx_v, rows0_v, acc_s, sem0):
    cid = lax.axis_index("c")
    sid = lax.axis_index("s")
    wid = sid * _NC + cid

    def init_blk(i, _):
        r0 = sid * _ROWS_PER_TILE + i * _CHUNK
        pltpu.sync_copy(y_hbm.at[pl.ds(r0, _CHUNK)], acc_s.at[pl.ds(r0, _CHUNK)])
        return 0

    lax.fori_loop(0, _ROWS_PER_TILE // _CHUNK, init_blk, 0)
    plsc.subcore_barrier()

    base = wid * _E_PER_W

    def body(i, _):
        off = base + i * _CHUNK
        pltpu.sync_copy(src_hbm.at[pl.ds(off, _CHUNK)], sidx_v)
        pltpu.sync_copy(dst_hbm.at[pl.ds(off, _CHUNK)], didx_v)
        pltpu.async_copy(y_hbm.at[sidx_v], rows0_v, sem0).wait()
        pltpu.sync_copy(rows0_v, acc_s.at[didx_v], add=True)
        return 0

    lax.fori_loop(0, _N_CHUNKS, body, 0)
    plsc.subcore_barrier()

    def out_blk(i, _):
        r0 = sid * _ROWS_PER_TILE + i * _CHUNK
        pltpu.sync_copy(acc_s.at[pl.ds(r0, _CHUNK)], out_hbm.at[cid, pl.ds(r0, _CHUNK)])
        return 0

    lax.fori_loop(0, _ROWS_PER_TILE // _CHUNK, out_blk, 0)


@functools.cache
def _agg_kernel():
    return pl.kernel(
        _agg_body,
        out_type=jax.ShapeDtypeStruct((_NC, _N_PAD, _D), jnp.float32),
        mesh=_sc_mesh(),
        scratch_types=[
            pltpu.VMEM((_CHUNK,), jnp.int32),
            pltpu.VMEM((_CHUNK,), jnp.int32),
            pltpu.VMEM((_CHUNK, _D), jnp.float32),
            pltpu.VMEM_SHARED((_N_PAD, _D), jnp.float32),
            pltpu.SemaphoreType.DMA,
        ],
    )


_BM = 512
_N_BLOCKS = _N_PAD // _BM


def _dis_block(degp):
    deg = degp[0, :, 0:1] + degp[1, :, 0:1] + 1.0
    return lax.rsqrt(deg)


def _b_kernel(x_ref, degp_ref, w1_ref, y1_ref):
    dis = _dis_block(degp_ref[...])
    y1_ref[...] = (x_ref[...] @ w1_ref[...]) * dis


def _d_kernel(aggp_ref, y1_ref, degp_ref, w2_ref, b1_ref, y2_ref):
    dis = _dis_block(degp_ref[...])
    agg = aggp_ref[0] + aggp_ref[1] - y1_ref[...]
    h1 = jnp.maximum(agg * dis + b1_ref[...], 0.0)
    y2_ref[...] = (h1 @ w2_ref[...]) * dis


def _f_kernel(aggp_ref, y2_ref, degp_ref, batch_ref, b2_ref, wf_ref, bf_ref,
              wo_ref, bo_ref, out_ref, sum_s, cnt_s):
    i = pl.program_id(0)
    dis = _dis_block(degp_ref[...])
    agg = aggp_ref[0] + aggp_ref[1] - y2_ref[...]
    h2 = agg * dis + b2_ref[...]

    b = batch_ref[0, 0, :].reshape(_BM, 1)
    gid = lax.broadcasted_iota(jnp.int32, (_BM, _G), 1)
    p = jnp.where(b == gid, 1.0, 0.0)

    contract = (((0,), (0,)), ((), ()))
    psum = lax.dot_general(p, h2, contract,
                           precision=lax.Precision.HIGHEST)
    pcnt = lax.dot_general(p, jnp.ones((_BM, _D), jnp.float32), contract)

    @pl.when(i == 0)
    def _():
        sum_s[...] = jnp.zeros_like(sum_s)
        cnt_s[...] = jnp.zeros_like(cnt_s)

    sum_s[...] += psum
    cnt_s[...] += pcnt

    @pl.when(i == _N_BLOCKS - 1)
    def _():
        pooled = sum_s[...] / jnp.maximum(cnt_s[...], 1.0)
        hf = jnp.maximum(pooled @ wf_ref[...] + bf_ref[...], 0.0)
        out_ref[...] = hf @ wo_ref[...] + bo_ref[...]


def _row_spec(width):
    return pl.BlockSpec((_BM, width), lambda i: (i, 0))


def _part_spec(width):
    return pl.BlockSpec((_NC, _BM, width), lambda i: (0, i, 0))


def _full_spec(shape):
    return pl.BlockSpec(shape, lambda i: tuple(0 for _ in shape))


_b_call = pl.pallas_call(
    _b_kernel,
    grid=(_N_BLOCKS,),
    in_specs=[_row_spec(_D), _part_spec(_DEG_W), _full_spec((_D, _D))],
    out_specs=_row_spec(_D),
    out_shape=jax.ShapeDtypeStruct((_N_PAD, _D), jnp.float32),
)

_d_call = pl.pallas_call(
    _d_kernel,
    grid=(_N_BLOCKS,),
    in_specs=[_part_spec(_D), _row_spec(_D), _part_spec(_DEG_W),
              _full_spec((_D, _D)), _full_spec((1, _D))],
    out_specs=_row_spec(_D),
    out_shape=jax.ShapeDtypeStruct((_N_PAD, _D), jnp.float32),
)

_f_call = pl.pallas_call(
    _f_kernel,
    grid=(_N_BLOCKS,),
    in_specs=[_part_spec(_D), _row_spec(_D), _part_spec(_DEG_W),
              pl.BlockSpec((1, 1, _BM), lambda i: (i, 0, 0)),
              _full_spec((1, _D)), _full_spec((_D, 64)), _full_spec((1, 64)),
              _full_spec((64, _D)), _full_spec((1, _D))],
    out_specs=_full_spec((_G, _D)),
    out_shape=jax.ShapeDtypeStruct((_G, _D), jnp.float32),
    scratch_shapes=[pltpu.VMEM((_G, _D), jnp.float32),
                    pltpu.VMEM((_G, _D), jnp.float32)],
)


@jax.jit
def kernel(x, edge_index, batch, W1, b1, W2, b2, Wf, bf, Wo, bo):
    src = jnp.pad(edge_index[0], (0, _E_PAD - _E), constant_values=_N)
    dst = jnp.pad(edge_index[1], (0, _E_PAD - _E), constant_values=_N)
    x_pad = jnp.pad(x, ((0, _N_PAD - _N), (0, 0)))
    batch_pad = jnp.pad(batch, (0, _N_PAD - _N), constant_values=_G)
    batch_pad = batch_pad.reshape(_N_BLOCKS, 1, _BM)

    onehot_rows = jnp.zeros((2, _CHUNK, _DEG_W), jnp.float32).at[0, :, 0].set(1.0)
    degp = _deg_kernel()(dst, onehot_rows)
    y1 = _b_call(x_pad, degp, W1)
    agg1 = _agg_kernel()(y1, src, dst)
    y2 = _d_call(agg1, y1, degp, W2, b1.reshape(1, _D))
    agg2 = _agg_kernel()(y2, src, dst)

    wo_pad = jnp.pad(Wo, ((0, 0), (0, _D - 1)))
    bo_pad = jnp.pad(bo, (0, _D - 1)).reshape(1, _D)
    out = _f_call(agg2, y2, degp, batch_pad, b2.reshape(1, _D),
                  Wf, bf.reshape(1, 64), wo_pad, bo_pad)
    return out[:, 0:1]


# speedup vs baseline: 1.3959x; 1.1857x over previous
import functools

import jax
import jax.numpy as jnp
from jax import lax
from jax.experimental import pallas as pl
from jax.experimental.pallas import tpu as pltpu
from jax.experimental.pallas import tpu_sc as plsc

_N = 10000
_E = 640000
_D = 128
_G = 16

_NC = 2
_NS = 16
_NW = _NC * _NS

_CHUNK = 128
_N_PAD = 10240
_ROWS_PER_TILE = _N_PAD // _NS
_N_CHUNKS = 157
_E_PER_W = _N_CHUNKS * _CHUNK
_E_PAD = _E_PER_W * _NW
_DEG_W = 128


def _sc_mesh():
    return plsc.VectorSubcoreMesh(
        core_axis_name="c", subcore_axis_name="s",
        num_cores=_NC, num_subcores=_NS,
    )


def _deg_body(dst_hbm, ones_hbm, out_hbm, ones_v, zeros_v, idx_v, acc_s):
    cid = lax.axis_index("c")
    sid = lax.axis_index("s")
    wid = sid * _NC + cid

    pltpu.sync_copy(ones_hbm.at[0], ones_v)
    pltpu.sync_copy(ones_hbm.at[1], zeros_v)

    def zero_blk(i, _):
        pltpu.sync_copy(
            zeros_v,
            acc_s.at[pl.ds(sid * _ROWS_PER_TILE + i * _CHUNK, _CHUNK)],
        )
        return 0

    lax.fori_loop(0, _ROWS_PER_TILE // _CHUNK, zero_blk, 0)
    plsc.subcore_barrier()

    base = wid * _E_PER_W

    def body(i, _):
        pltpu.sync_copy(dst_hbm.at[pl.ds(base + i * _CHUNK, _CHUNK)], idx_v)
        pltpu.sync_copy(ones_v, acc_s.at[idx_v], add=True)
        return 0

    lax.fori_loop(0, _N_CHUNKS, body, 0)
    plsc.subcore_barrier()

    pltpu.sync_copy(
        acc_s.at[pl.ds(sid * _ROWS_PER_TILE, _ROWS_PER_TILE)],
        out_hbm.at[cid, pl.ds(sid * _ROWS_PER_TILE, _ROWS_PER_TILE)],
    )


@functools.cache
def _deg_kernel():
    return pl.kernel(
        _deg_body,
        out_type=jax.ShapeDtypeStruct((_NC, _N_PAD, _DEG_W), jnp.float32),
        mesh=_sc_mesh(),
        scratch_types=[
            pltpu.VMEM((_CHUNK, _DEG_W), jnp.float32),
            pltpu.VMEM((_CHUNK, _DEG_W), jnp.float32),
            pltpu.VMEM((_CHUNK,), jnp.int32),
            pltpu.VMEM_SHARED((_N_PAD, _DEG_W), jnp.float32),
        ],
    )


def _agg_body(y_hbm, src_hbm, dst_hbm, out_hbm, sidx_v, didx_v, rows0_v, acc_s, sem0):
    cid = lax.axis_index("c")
    sid = lax.axis_index("s")
    wid = sid * _NC + cid

    def init_blk(i, _):
        r0 = sid * _ROWS_PER_TILE + i * _CHUNK
        pltpu.sync_copy(y_hbm.at[pl.ds(r0, _CHUNK)], acc_s.at[pl.ds(r0, _CHUNK)])
        return 0

    lax.fori_loop(0, _ROWS_PER_TILE // _CHUNK, init_blk, 0)
    plsc.subcore_barrier()

    base = wid * _E_PER_W

    def body(i, _):
        off = base + i * _CHUNK
        pltpu.sync_copy(src_hbm.at[pl.ds(off, _CHUNK)], sidx_v)
        pltpu.sync_copy(dst_hbm.at[pl.ds(off, _CHUNK)], didx_v)
        pltpu.async_copy(y_hbm.at[sidx_v], rows0_v, sem0).wait()
        pltpu.sync_copy(rows0_v, acc_s.at[didx_v], add=True)
        return 0

    lax.fori_loop(0, _N_CHUNKS, body, 0)
    plsc.subcore_barrier()

    def out_blk(i, _):
        r0 = sid * _ROWS_PER_TILE + i * _CHUNK
        pltpu.sync_copy(acc_s.at[pl.ds(r0, _CHUNK)], out_hbm.at[cid, pl.ds(r0, _CHUNK)])
        return 0

    lax.fori_loop(0, _ROWS_PER_TILE // _CHUNK, out_blk, 0)


@functools.cache
def _agg_kernel():
    return pl.kernel(
        _agg_body,
        out_type=jax.ShapeDtypeStruct((_NC, _N_PAD, _D), jnp.float32),
        mesh=_sc_mesh(),
        scratch_types=[
            pltpu.VMEM((_CHUNK,), jnp.int32),
            pltpu.VMEM((_CHUNK,), jnp.int32),
            pltpu.VMEM((_CHUNK, _D), jnp.float32),
            pltpu.VMEM_SHARED((_N_PAD, _D), jnp.float32),
            pltpu.SemaphoreType.DMA,
        ],
    )


_BM = 512
_N_BLOCKS = _N_PAD // _BM


def _dis_block(degp):
    deg = degp[0, :, 0:1] + degp[1, :, 0:1] + 1.0
    return lax.rsqrt(deg)


def _b_kernel(x_ref, degp_ref, w1_ref, y1_ref):
    dis = _dis_block(degp_ref[...])
    y1_ref[...] = (x_ref[...] @ w1_ref[...]) * dis


def _d_kernel(aggp_ref, y1_ref, degp_ref, w2_ref, b1_ref, y2_ref):
    dis = _dis_block(degp_ref[...])
    agg = aggp_ref[0] + aggp_ref[1] - y1_ref[...]
    h1 = jnp.maximum(agg * dis + b1_ref[...], 0.0)
    y2_ref[...] = (h1 @ w2_ref[...]) * dis


def _f_kernel(aggp_ref, y2_ref, degp_ref, batch_ref, b2_ref, wf_ref, bf_ref,
              wo_ref, bo_ref, out_ref, sum_s, cnt_s):
    i = pl.program_id(0)
    dis = _dis_block(degp_ref[...])
    agg = aggp_ref[0] + aggp_ref[1] - y2_ref[...]
    h2 = agg * dis + b2_ref[...]

    b = batch_ref[0, 0, :].reshape(_BM, 1)
    gid = lax.broadcasted_iota(jnp.int32, (_BM, _G), 1)
    p = jnp.where(b == gid, 1.0, 0.0)

    contract = (((0,), (0,)), ((), ()))
    psum = lax.dot_general(p, h2, contract,
                           precision=lax.Precision.HIGHEST)
    pcnt = lax.dot_general(p, jnp.ones((_BM, _D), jnp.float32), contract)

    @pl.when(i == 0)
    def _():
        sum_s[...] = jnp.zeros_like(sum_s)
        cnt_s[...] = jnp.zeros_like(cnt_s)

    sum_s[...] += psum
    cnt_s[...] += pcnt

    @pl.when(i == _N_BLOCKS - 1)
    def _():
        pooled = sum_s[...] / jnp.maximum(cnt_s[...], 1.0)
        hf = jnp.maximum(pooled @ wf_ref[...] + bf_ref[...], 0.0)
        out_ref[...] = hf @ wo_ref[...] + bo_ref[...]


def _row_spec(width):
    return pl.BlockSpec((_BM, width), lambda i: (i, 0))


def _part_spec(width):
    return pl.BlockSpec((_NC, _BM, width), lambda i: (0, i, 0))


def _full_spec(shape):
    return pl.BlockSpec(shape, lambda i: tuple(0 for _ in shape))


_b_call = pl.pallas_call(
    _b_kernel,
    grid=(_N_BLOCKS,),
    in_specs=[_row_spec(_D), _part_spec(_DEG_W), _full_spec((_D, _D))],
    out_specs=_row_spec(_D),
    out_shape=jax.ShapeDtypeStruct((_N_PAD, _D), jnp.float32),
)

_d_call = pl.pallas_call(
    _d_kernel,
    grid=(_N_BLOCKS,),
    in_specs=[_part_spec(_D), _row_spec(_D), _part_spec(_DEG_W),
              _full_spec((_D, _D)), _full_spec((1, _D))],
    out_specs=_row_spec(_D),
    out_shape=jax.ShapeDtypeStruct((_N_PAD, _D), jnp.float32),
)

_f_call = pl.pallas_call(
    _f_kernel,
    grid=(_N_BLOCKS,),
    in_specs=[_part_spec(_D), _row_spec(_D), _part_spec(_DEG_W),
              pl.BlockSpec((1, 1, _BM), lambda i: (i, 0, 0)),
              _full_spec((1, _D)), _full_spec((_D, 64)), _full_spec((1, 64)),
              _full_spec((64, _D)), _full_spec((1, _D))],
    out_specs=_full_spec((_G, _D)),
    out_shape=jax.ShapeDtypeStruct((_G, _D), jnp.float32),
    scratch_shapes=[pltpu.VMEM((_G, _D), jnp.float32),
                    pltpu.VMEM((_G, _D), jnp.float32)],
)


@jax.jit
def kernel(x, edge_index, batch, W1, b1, W2, b2, Wf, bf, Wo, bo):
    src = jnp.pad(edge_index[0], (0, _E_PAD - _E), constant_values=_N)
    dst = jnp.pad(edge_index[1], (0, _E_PAD - _E), constant_values=_N)
    x_pad = jnp.pad(x, ((0, _N_PAD - _N), (0, 0)))
    batch_pad = jnp.pad(batch, (0, _N_PAD - _N), constant_values=_G)
    batch_pad = batch_pad.reshape(_N_BLOCKS, 1, _BM)

    onehot_rows = jnp.zeros((2, _CHUNK, _DEG_W), jnp.float32).at[0, :, 0].set(1.0)
    degp = _deg_kernel()(dst, onehot_rows)
    y1 = _b_call(x_pad, degp, W1)
    agg1 = _agg_kernel()(y1, src, dst)
    y2 = _d_call(agg1, y1, degp, W2, b1.reshape(1, _D))
    agg2 = _agg_kernel()(y2, src, dst)

    wo_pad = jnp.pad(Wo, ((0, 0), (0, _D - 1)))
    bo_pad = jnp.pad(bo, (0, _D - 1)).reshape(1, _D)
    out = _f_call(agg2, y2, degp, batch_pad, b2.reshape(1, _D),
                  Wf, bf.reshape(1, 64), wo_pad, bo_pad)
    return out[:, 0:1]


# speedup vs baseline: 1.6794x; 1.2030x over previous
import functools

import jax
import jax.numpy as jnp
from jax import lax
from jax.experimental import pallas as pl
from jax.experimental.pallas import tpu as pltpu
from jax.experimental.pallas import tpu_sc as plsc

_N = 10000
_E = 640000
_D = 128
_G = 16

_NC = 2
_NS = 16
_NW = _NC * _NS

_CHUNK = 128
_N_PAD = 10240
_ROWS_PER_TILE = _N_PAD // _NS
_N_CHUNKS = 157
_E_PER_W = _N_CHUNKS * _CHUNK
_E_PAD = _E_PER_W * _NW
_DEG_W = 128


def _sc_mesh():
    return plsc.VectorSubcoreMesh(
        core_axis_name="c", subcore_axis_name="s",
        num_cores=_NC, num_subcores=_NS,
    )


def _deg_body(dst_hbm, ones_hbm, out_hbm, ones_v, zeros_v, idx_v, acc_s):
    cid = lax.axis_index("c")
    sid = lax.axis_index("s")
    wid = sid * _NC + cid

    pltpu.sync_copy(ones_hbm.at[0], ones_v)
    pltpu.sync_copy(ones_hbm.at[1], zeros_v)

    def zero_blk(i, _):
        pltpu.sync_copy(
            zeros_v,
            acc_s.at[pl.ds(sid * _ROWS_PER_TILE + i * _CHUNK, _CHUNK)],
        )
        return 0

    lax.fori_loop(0, _ROWS_PER_TILE // _CHUNK, zero_blk, 0)
    plsc.subcore_barrier()

    base = wid * _E_PER_W

    def body(i, _):
        pltpu.sync_copy(dst_hbm.at[pl.ds(base + i * _CHUNK, _CHUNK)], idx_v)
        pltpu.sync_copy(ones_v, acc_s.at[idx_v], add=True)
        return 0

    lax.fori_loop(0, _N_CHUNKS, body, 0)
    plsc.subcore_barrier()

    pltpu.sync_copy(
        acc_s.at[pl.ds(sid * _ROWS_PER_TILE, _ROWS_PER_TILE)],
        out_hbm.at[cid, pl.ds(sid * _ROWS_PER_TILE, _ROWS_PER_TILE)],
    )


@functools.cache
def _deg_kernel():
    return pl.kernel(
        _deg_body,
        out_type=jax.ShapeDtypeStruct((_NC, _N_PAD, _DEG_W), jnp.float32),
        mesh=_sc_mesh(),
        scratch_types=[
            pltpu.VMEM((_CHUNK, _DEG_W), jnp.float32),
            pltpu.VMEM((_CHUNK, _DEG_W), jnp.float32),
            pltpu.VMEM((_CHUNK,), jnp.int32),
            pltpu.VMEM_SHARED((_N_PAD, _DEG_W), jnp.float32),
        ],
    )


def _agg_body(y_hbm, src_hbm, dst_hbm, out_hbm, sidx_v, didx_v, rows0_v, acc_s, sem0):
    cid = lax.axis_index("c")
    sid = lax.axis_index("s")
    wid = sid * _NC + cid

    def init_blk(i, _):
        r0 = sid * _ROWS_PER_TILE + i * _CHUNK
        pltpu.sync_copy(y_hbm.at[pl.ds(r0, _CHUNK)], acc_s.at[pl.ds(r0, _CHUNK)])
        return 0

    lax.fori_loop(0, _ROWS_PER_TILE // _CHUNK, init_blk, 0)
    plsc.subcore_barrier()

    base = wid * _E_PER_W

    def body(i, _):
        off = base + i * _CHUNK
        pltpu.sync_copy(src_hbm.at[pl.ds(off, _CHUNK)], sidx_v)
        pltpu.sync_copy(dst_hbm.at[pl.ds(off, _CHUNK)], didx_v)
        pltpu.async_copy(y_hbm.at[sidx_v], rows0_v, sem0).wait()
        pltpu.sync_copy(rows0_v, acc_s.at[didx_v], add=True)
        return 0

    lax.fori_loop(0, _N_CHUNKS, body, 0)
    plsc.subcore_barrier()

    def out_blk(i, _):
        r0 = sid * _ROWS_PER_TILE + i * _CHUNK
        pltpu.sync_copy(acc_s.at[pl.ds(r0, _CHUNK)], out_hbm.at[cid, pl.ds(r0, _CHUNK)])
        return 0

    lax.fori_loop(0, _ROWS_PER_TILE // _CHUNK, out_blk, 0)


@functools.cache
def _agg_kernel():
    return pl.kernel(
        _agg_body,
        out_type=jax.ShapeDtypeStruct((_NC, _N_PAD, _D), jnp.float32),
        mesh=_sc_mesh(),
        scratch_types=[
            pltpu.VMEM((_CHUNK,), jnp.int32),
            pltpu.VMEM((_CHUNK,), jnp.int32),
            pltpu.VMEM((_CHUNK, _D), jnp.float32),
            pltpu.VMEM_SHARED((_N_PAD, _D), jnp.float32),
            pltpu.SemaphoreType.DMA,
        ],
    )


_BM = 512
_N_BLOCKS = _N_PAD // _BM


def _dis_block(degp):
    deg = degp[0, :, 0:1] + degp[1, :, 0:1] + 1.0
    return lax.rsqrt(deg)


def _b_kernel(x_ref, degp_ref, w1_ref, y1_ref):
    dis = _dis_block(degp_ref[...])
    y1_ref[...] = (x_ref[...] @ w1_ref[...]) * dis


def _d_kernel(aggp_ref, y1_ref, degp_ref, w2_ref, b1_ref, y2_ref):
    dis = _dis_block(degp_ref[...])
    agg = aggp_ref[0] + aggp_ref[1] - y1_ref[...]
    h1 = jnp.maximum(agg * dis + b1_ref[...], 0.0)
    y2_ref[...] = (h1 @ w2_ref[...]) * dis


def _f_kernel(aggp_ref, y2_ref, degp_ref, batch_ref, b2_ref, wf_ref, bf_ref,
              wo_ref, bo_ref, out_ref, sum_s, cnt_s):
    i = pl.program_id(0)
    dis = _dis_block(degp_ref[...])
    agg = aggp_ref[0] + aggp_ref[1] - y2_ref[...]
    h2 = agg * dis + b2_ref[...]

    b = batch_ref[0, 0, :].reshape(_BM, 1)
    gid = lax.broadcasted_iota(jnp.int32, (_BM, _G), 1)
    p = jnp.where(b == gid, 1.0, 0.0)

    contract = (((0,), (0,)), ((), ()))
    psum = lax.dot_general(p, h2, contract,
                           precision=lax.Precision.HIGHEST)
    pcnt = lax.dot_general(p, jnp.ones((_BM, _D), jnp.float32), contract)

    @pl.when(i == 0)
    def _():
        sum_s[...] = jnp.zeros_like(sum_s)
        cnt_s[...] = jnp.zeros_like(cnt_s)

    sum_s[...] += psum
    cnt_s[...] += pcnt

    @pl.when(i == _N_BLOCKS - 1)
    def _():
        pooled = sum_s[...] / jnp.maximum(cnt_s[...], 1.0)
        hf = jnp.maximum(pooled @ wf_ref[...] + bf_ref[...], 0.0)
        out_ref[...] = hf @ wo_ref[...] + bo_ref[...]


def _row_spec(width):
    return pl.BlockSpec((_BM, width), lambda i: (i, 0))


def _part_spec(width):
    return pl.BlockSpec((_NC, _BM, width), lambda i: (0, i, 0))


def _full_spec(shape):
    return pl.BlockSpec(shape, lambda i: tuple(0 for _ in shape))


_b_call = pl.pallas_call(
    _b_kernel,
    grid=(_N_BLOCKS,),
    in_specs=[_row_spec(_D), _part_spec(_DEG_W), _full_spec((_D, _D))],
    out_specs=_row_spec(_D),
    out_shape=jax.ShapeDtypeStruct((_N_PAD, _D), jnp.float32),
)

_d_call = pl.pallas_call(
    _d_kernel,
    grid=(_N_BLOCKS,),
    in_specs=[_part_spec(_D), _row_spec(_D), _part_spec(_DEG_W),
              _full_spec((_D, _D)), _full_spec((1, _D))],
    out_specs=_row_spec(_D),
    out_shape=jax.ShapeDtypeStruct((_N_PAD, _D), jnp.float32),
)

_f_call = pl.pallas_call(
    _f_kernel,
    grid=(_N_BLOCKS,),
    in_specs=[_part_spec(_D), _row_spec(_D), _part_spec(_DEG_W),
              pl.BlockSpec((1, 1, _BM), lambda i: (i, 0, 0)),
              _full_spec((1, _D)), _full_spec((_D, 64)), _full_spec((1, 64)),
              _full_spec((64, _D)), _full_spec((1, _D))],
    out_specs=_full_spec((_G, _D)),
    out_shape=jax.ShapeDtypeStruct((_G, _D), jnp.float32),
    scratch_shapes=[pltpu.VMEM((_G, _D), jnp.float32),
                    pltpu.VMEM((_G, _D), jnp.float32)],
)


@jax.jit
def kernel(x, edge_index, batch, W1, b1, W2, b2, Wf, bf, Wo, bo):
    padv = _N + jnp.arange(_E_PAD - _E, dtype=jnp.int32) % (_N_PAD - _N)
    src = jnp.concatenate([edge_index[0], padv])
    dst = jnp.concatenate([edge_index[1], padv])
    x_pad = jnp.pad(x, ((0, _N_PAD - _N), (0, 0)))
    batch_pad = jnp.pad(batch, (0, _N_PAD - _N), constant_values=_G)
    batch_pad = batch_pad.reshape(_N_BLOCKS, 1, _BM)

    onehot_rows = jnp.zeros((2, _CHUNK, _DEG_W), jnp.float32).at[0, :, 0].set(1.0)
    degp = _deg_kernel()(dst, onehot_rows)
    y1 = _b_call(x_pad, degp, W1)
    agg1 = _agg_kernel()(y1, src, dst)
    y2 = _d_call(agg1, y1, degp, W2, b1.reshape(1, _D))
    agg2 = _agg_kernel()(y2, src, dst)

    wo_pad = jnp.pad(Wo, ((0, 0), (0, _D - 1)))
    bo_pad = jnp.pad(bo, (0, _D - 1)).reshape(1, _D)
    out = _f_call(agg2, y2, degp, batch_pad, b2.reshape(1, _D),
                  Wf, bf.reshape(1, 64), wo_pad, bo_pad)
    return out[:, 0:1]


# speedup vs baseline: 1.7433x; 1.0381x over previous
import functools

import jax
import jax.numpy as jnp
from jax import lax
from jax.experimental import pallas as pl
from jax.experimental.pallas import tpu as pltpu
from jax.experimental.pallas import tpu_sc as plsc

_N = 10000
_E = 640000
_D = 128
_G = 16

_NC = 2
_NS = 16
_NW = _NC * _NS

_CHUNK = 128
_N_PAD = 10240
_ROWS_PER_TILE = _N_PAD // _NS
_N_CHUNKS = 157
_E_PER_W = _N_CHUNKS * _CHUNK
_E_PAD = _E_PER_W * _NW
_DEG_W = 64


def _sc_mesh():
    return plsc.VectorSubcoreMesh(
        core_axis_name="c", subcore_axis_name="s",
        num_cores=_NC, num_subcores=_NS,
    )


def _deg_body(dst_hbm, ones_hbm, out_hbm, ones_v, zeros_v, idx_v, acc_s):
    cid = lax.axis_index("c")
    sid = lax.axis_index("s")
    wid = sid * _NC + cid

    pltpu.sync_copy(ones_hbm.at[0], ones_v)
    pltpu.sync_copy(ones_hbm.at[1], zeros_v)

    def zero_blk(i, _):
        pltpu.sync_copy(
            zeros_v,
            acc_s.at[pl.ds(sid * _ROWS_PER_TILE + i * _CHUNK, _CHUNK)],
        )
        return 0

    lax.fori_loop(0, _ROWS_PER_TILE // _CHUNK, zero_blk, 0)
    plsc.subcore_barrier()

    base = wid * _E_PER_W

    def body(i, _):
        pltpu.sync_copy(dst_hbm.at[pl.ds(base + i * _CHUNK, _CHUNK)], idx_v)
        pltpu.sync_copy(ones_v, acc_s.at[idx_v], add=True)
        return 0

    lax.fori_loop(0, _N_CHUNKS, body, 0)
    plsc.subcore_barrier()

    pltpu.sync_copy(
        acc_s.at[pl.ds(sid * _ROWS_PER_TILE, _ROWS_PER_TILE)],
        out_hbm.at[cid, pl.ds(sid * _ROWS_PER_TILE, _ROWS_PER_TILE)],
    )


@functools.cache
def _deg_kernel():
    return pl.kernel(
        _deg_body,
        out_type=jax.ShapeDtypeStruct((_NC, _N_PAD, _DEG_W), jnp.float32),
        mesh=_sc_mesh(),
        scratch_types=[
            pltpu.VMEM((_CHUNK, _DEG_W), jnp.float32),
            pltpu.VMEM((_CHUNK, _DEG_W), jnp.float32),
            pltpu.VMEM((_CHUNK,), jnp.int32),
            pltpu.VMEM_SHARED((_N_PAD, _DEG_W), jnp.float32),
        ],
    )


def _agg_body(y_hbm, src_hbm, dst_hbm, out_hbm, sidx_v, didx_v, rows0_v, acc_s, sem0):
    cid = lax.axis_index("c")
    sid = lax.axis_index("s")
    wid = sid * _NC + cid

    def init_blk(i, _):
        r0 = sid * _ROWS_PER_TILE + i * _CHUNK
        pltpu.sync_copy(y_hbm.at[pl.ds(r0, _CHUNK)], acc_s.at[pl.ds(r0, _CHUNK)])
        return 0

    lax.fori_loop(0, _ROWS_PER_TILE // _CHUNK, init_blk, 0)
    plsc.subcore_barrier()

    base = wid * _E_PER_W

    def body(i, _):
        off = base + i * _CHUNK
        pltpu.sync_copy(src_hbm.at[pl.ds(off, _CHUNK)], sidx_v)
        pltpu.sync_copy(dst_hbm.at[pl.ds(off, _CHUNK)], didx_v)
        pltpu.async_copy(y_hbm.at[sidx_v], rows0_v, sem0).wait()
        pltpu.sync_copy(rows0_v, acc_s.at[didx_v], add=True)
        return 0

    lax.fori_loop(0, _N_CHUNKS, body, 0)
    plsc.subcore_barrier()

    def out_blk(i, _):
        r0 = sid * _ROWS_PER_TILE + i * _CHUNK
        pltpu.sync_copy(acc_s.at[pl.ds(r0, _CHUNK)], out_hbm.at[cid, pl.ds(r0, _CHUNK)])
        return 0

    lax.fori_loop(0, _ROWS_PER_TILE // _CHUNK, out_blk, 0)


@functools.cache
def _agg_kernel():
    return pl.kernel(
        _agg_body,
        out_type=jax.ShapeDtypeStruct((_NC, _N_PAD, _D), jnp.float32),
        mesh=_sc_mesh(),
        scratch_types=[
            pltpu.VMEM((_CHUNK,), jnp.int32),
            pltpu.VMEM((_CHUNK,), jnp.int32),
            pltpu.VMEM((_CHUNK, _D), jnp.float32),
            pltpu.VMEM_SHARED((_N_PAD, _D), jnp.float32),
            pltpu.SemaphoreType.DMA,
        ],
    )


_BM = 512
_N_BLOCKS = _N_PAD // _BM


def _dis_block(degp):
    deg = degp[0, :, 0:1] + degp[1, :, 0:1] + 1.0
    return lax.rsqrt(deg)


def _b_kernel(x_ref, degp_ref, w1_ref, y1_ref):
    dis = _dis_block(degp_ref[...])
    y1_ref[...] = (x_ref[...] @ w1_ref[...]) * dis


def _d_kernel(aggp_ref, y1_ref, degp_ref, w2_ref, b1_ref, y2_ref):
    dis = _dis_block(degp_ref[...])
    agg = aggp_ref[0] + aggp_ref[1] - y1_ref[...]
    h1 = jnp.maximum(agg * dis + b1_ref[...], 0.0)
    y2_ref[...] = (h1 @ w2_ref[...]) * dis


def _f_kernel(aggp_ref, y2_ref, degp_ref, batch_ref, b2_ref, wf_ref, bf_ref,
              wo_ref, bo_ref, out_ref, sum_s, cnt_s):
    i = pl.program_id(0)
    dis = _dis_block(degp_ref[...])
    agg = aggp_ref[0] + aggp_ref[1] - y2_ref[...]
    h2 = agg * dis + b2_ref[...]

    b = batch_ref[0, 0, :].reshape(_BM, 1)
    gid = lax.broadcasted_iota(jnp.int32, (_BM, _G), 1)
    p = jnp.where(b == gid, 1.0, 0.0)

    contract = (((0,), (0,)), ((), ()))
    psum = lax.dot_general(p, h2, contract,
                           precision=lax.Precision.HIGHEST)
    pcnt = lax.dot_general(p, jnp.ones((_BM, _D), jnp.float32), contract)

    @pl.when(i == 0)
    def _():
        sum_s[...] = jnp.zeros_like(sum_s)
        cnt_s[...] = jnp.zeros_like(cnt_s)

    sum_s[...] += psum
    cnt_s[...] += pcnt

    @pl.when(i == _N_BLOCKS - 1)
    def _():
        pooled = sum_s[...] / jnp.maximum(cnt_s[...], 1.0)
        hf = jnp.maximum(pooled @ wf_ref[...] + bf_ref[...], 0.0)
        out_ref[...] = hf @ wo_ref[...] + bo_ref[...]


def _row_spec(width):
    return pl.BlockSpec((_BM, width), lambda i: (i, 0))


def _part_spec(width):
    return pl.BlockSpec((_NC, _BM, width), lambda i: (0, i, 0))


def _full_spec(shape):
    return pl.BlockSpec(shape, lambda i: tuple(0 for _ in shape))


_b_call = pl.pallas_call(
    _b_kernel,
    grid=(_N_BLOCKS,),
    in_specs=[_row_spec(_D), _part_spec(_DEG_W), _full_spec((_D, _D))],
    out_specs=_row_spec(_D),
    out_shape=jax.ShapeDtypeStruct((_N_PAD, _D), jnp.float32),
)

_d_call = pl.pallas_call(
    _d_kernel,
    grid=(_N_BLOCKS,),
    in_specs=[_part_spec(_D), _row_spec(_D), _part_spec(_DEG_W),
              _full_spec((_D, _D)), _full_spec((1, _D))],
    out_specs=_row_spec(_D),
    out_shape=jax.ShapeDtypeStruct((_N_PAD, _D), jnp.float32),
)

_f_call = pl.pallas_call(
    _f_kernel,
    grid=(_N_BLOCKS,),
    in_specs=[_part_spec(_D), _row_spec(_D), _part_spec(_DEG_W),
              pl.BlockSpec((1, 1, _BM), lambda i: (i, 0, 0)),
              _full_spec((1, _D)), _full_spec((_D, 64)), _full_spec((1, 64)),
              _full_spec((64, _D)), _full_spec((1, _D))],
    out_specs=_full_spec((_G, _D)),
    out_shape=jax.ShapeDtypeStruct((_G, _D), jnp.float32),
    scratch_shapes=[pltpu.VMEM((_G, _D), jnp.float32),
                    pltpu.VMEM((_G, _D), jnp.float32)],
)


@jax.jit
def kernel(x, edge_index, batch, W1, b1, W2, b2, Wf, bf, Wo, bo):
    padv = _N + jnp.arange(_E_PAD - _E, dtype=jnp.int32) % (_N_PAD - _N)
    src = jnp.concatenate([edge_index[0], padv])
    dst = jnp.concatenate([edge_index[1], padv])
    x_pad = jnp.pad(x, ((0, _N_PAD - _N), (0, 0)))
    batch_pad = jnp.pad(batch, (0, _N_PAD - _N), constant_values=_G)
    batch_pad = batch_pad.reshape(_N_BLOCKS, 1, _BM)

    onehot_rows = jnp.zeros((2, _CHUNK, _DEG_W), jnp.float32).at[0, :, 0].set(1.0)
    degp = _deg_kernel()(dst, onehot_rows)
    y1 = _b_call(x_pad, degp, W1)
    agg1 = _agg_kernel()(y1, src, dst)
    y2 = _d_call(agg1, y1, degp, W2, b1.reshape(1, _D))
    agg2 = _agg_kernel()(y2, src, dst)

    wo_pad = jnp.pad(Wo, ((0, 0), (0, _D - 1)))
    bo_pad = jnp.pad(bo, (0, _D - 1)).reshape(1, _D)
    out = _f_call(agg2, y2, degp, batch_pad, b2.reshape(1, _D),
                  Wf, bf.reshape(1, 64), wo_pad, bo_pad)
    return out[:, 0:1]


# speedup vs baseline: 1.7782x; 1.0200x over previous
import functools

import jax
import jax.numpy as jnp
from jax import lax
from jax.experimental import pallas as pl
from jax.experimental.pallas import tpu as pltpu
from jax.experimental.pallas import tpu_sc as plsc

_N = 10000
_E = 640000
_D = 128
_G = 16

_NC = 2
_NS = 16
_NW = _NC * _NS

_CHUNK = 128
_N_PAD = 10240
_ROWS_PER_TILE = _N_PAD // _NS
_N_CHUNKS = 157
_E_PER_W = _N_CHUNKS * _CHUNK
_E_PAD = _E_PER_W * _NW
_DEG_W = 32


def _sc_mesh():
    return plsc.VectorSubcoreMesh(
        core_axis_name="c", subcore_axis_name="s",
        num_cores=_NC, num_subcores=_NS,
    )


def _deg_body(dst_hbm, ones_hbm, out_hbm, ones_v, zeros_v, idx_v, acc_s):
    cid = lax.axis_index("c")
    sid = lax.axis_index("s")
    wid = sid * _NC + cid

    pltpu.sync_copy(ones_hbm.at[0], ones_v)
    pltpu.sync_copy(ones_hbm.at[1], zeros_v)

    def zero_blk(i, _):
        pltpu.sync_copy(
            zeros_v,
            acc_s.at[pl.ds(sid * _ROWS_PER_TILE + i * _CHUNK, _CHUNK)],
        )
        return 0

    lax.fori_loop(0, _ROWS_PER_TILE // _CHUNK, zero_blk, 0)
    plsc.subcore_barrier()

    base = wid * _E_PER_W

    def body(i, _):
        pltpu.sync_copy(dst_hbm.at[pl.ds(base + i * _CHUNK, _CHUNK)], idx_v)
        pltpu.sync_copy(ones_v, acc_s.at[idx_v], add=True)
        return 0

    lax.fori_loop(0, _N_CHUNKS, body, 0)
    plsc.subcore_barrier()

    pltpu.sync_copy(
        acc_s.at[pl.ds(sid * _ROWS_PER_TILE, _ROWS_PER_TILE)],
        out_hbm.at[cid, pl.ds(sid * _ROWS_PER_TILE, _ROWS_PER_TILE)],
    )


@functools.cache
def _deg_kernel():
    return pl.kernel(
        _deg_body,
        out_type=jax.ShapeDtypeStruct((_NC, _N_PAD, _DEG_W), jnp.float32),
        mesh=_sc_mesh(),
        scratch_types=[
            pltpu.VMEM((_CHUNK, _DEG_W), jnp.float32),
            pltpu.VMEM((_CHUNK, _DEG_W), jnp.float32),
            pltpu.VMEM((_CHUNK,), jnp.int32),
            pltpu.VMEM_SHARED((_N_PAD, _DEG_W), jnp.float32),
        ],
    )


def _agg_body(y_hbm, src_hbm, dst_hbm, out_hbm, sidx_v, didx_v, rows0_v, acc_s, sem0):
    cid = lax.axis_index("c")
    sid = lax.axis_index("s")
    wid = sid * _NC + cid

    def init_blk(i, _):
        r0 = sid * _ROWS_PER_TILE + i * _CHUNK
        pltpu.sync_copy(y_hbm.at[pl.ds(r0, _CHUNK)], acc_s.at[pl.ds(r0, _CHUNK)])
        return 0

    lax.fori_loop(0, _ROWS_PER_TILE // _CHUNK, init_blk, 0)
    plsc.subcore_barrier()

    base = wid * _E_PER_W

    def body(i, _):
        off = base + i * _CHUNK
        pltpu.sync_copy(src_hbm.at[pl.ds(off, _CHUNK)], sidx_v)
        pltpu.sync_copy(dst_hbm.at[pl.ds(off, _CHUNK)], didx_v)
        pltpu.async_copy(y_hbm.at[sidx_v], rows0_v, sem0).wait()
        pltpu.sync_copy(rows0_v, acc_s.at[didx_v], add=True)
        return 0

    lax.fori_loop(0, _N_CHUNKS, body, 0)
    plsc.subcore_barrier()

    def out_blk(i, _):
        r0 = sid * _ROWS_PER_TILE + i * _CHUNK
        pltpu.sync_copy(acc_s.at[pl.ds(r0, _CHUNK)], out_hbm.at[cid, pl.ds(r0, _CHUNK)])
        return 0

    lax.fori_loop(0, _ROWS_PER_TILE // _CHUNK, out_blk, 0)


@functools.cache
def _agg_kernel():
    return pl.kernel(
        _agg_body,
        out_type=jax.ShapeDtypeStruct((_NC, _N_PAD, _D), jnp.float32),
        mesh=_sc_mesh(),
        scratch_types=[
            pltpu.VMEM((_CHUNK,), jnp.int32),
            pltpu.VMEM((_CHUNK,), jnp.int32),
            pltpu.VMEM((_CHUNK, _D), jnp.float32),
            pltpu.VMEM_SHARED((_N_PAD, _D), jnp.float32),
            pltpu.SemaphoreType.DMA,
        ],
    )


_BM = 512
_N_BLOCKS = _N_PAD // _BM


def _dis_block(degp):
    deg = degp[0, :, 0:1] + degp[1, :, 0:1] + 1.0
    return lax.rsqrt(deg)


def _b_kernel(x_ref, degp_ref, w1_ref, y1_ref):
    dis = _dis_block(degp_ref[...])
    y1_ref[...] = (x_ref[...] @ w1_ref[...]) * dis


def _d_kernel(aggp_ref, y1_ref, degp_ref, w2_ref, b1_ref, y2_ref):
    dis = _dis_block(degp_ref[...])
    agg = aggp_ref[0] + aggp_ref[1] - y1_ref[...]
    h1 = jnp.maximum(agg * dis + b1_ref[...], 0.0)
    y2_ref[...] = (h1 @ w2_ref[...]) * dis


def _f_kernel(aggp_ref, y2_ref, degp_ref, batch_ref, b2_ref, wf_ref, bf_ref,
              wo_ref, bo_ref, out_ref, sum_s, cnt_s):
    i = pl.program_id(0)
    dis = _dis_block(degp_ref[...])
    agg = aggp_ref[0] + aggp_ref[1] - y2_ref[...]
    h2 = agg * dis + b2_ref[...]

    b = batch_ref[0, 0, :].reshape(_BM, 1)
    gid = lax.broadcasted_iota(jnp.int32, (_BM, _G), 1)
    p = jnp.where(b == gid, 1.0, 0.0)

    contract = (((0,), (0,)), ((), ()))
    psum = lax.dot_general(p, h2, contract,
                           precision=lax.Precision.HIGHEST)
    pcnt = lax.dot_general(p, jnp.ones((_BM, _D), jnp.float32), contract)

    @pl.when(i == 0)
    def _():
        sum_s[...] = jnp.zeros_like(sum_s)
        cnt_s[...] = jnp.zeros_like(cnt_s)

    sum_s[...] += psum
    cnt_s[...] += pcnt

    @pl.when(i == _N_BLOCKS - 1)
    def _():
        pooled = sum_s[...] / jnp.maximum(cnt_s[...], 1.0)
        hf = jnp.maximum(pooled @ wf_ref[...] + bf_ref[...], 0.0)
        out_ref[...] = hf @ wo_ref[...] + bo_ref[...]


def _row_spec(width):
    return pl.BlockSpec((_BM, width), lambda i: (i, 0))


def _part_spec(width):
    return pl.BlockSpec((_NC, _BM, width), lambda i: (0, i, 0))


def _full_spec(shape):
    return pl.BlockSpec(shape, lambda i: tuple(0 for _ in shape))


_b_call = pl.pallas_call(
    _b_kernel,
    grid=(_N_BLOCKS,),
    in_specs=[_row_spec(_D), _part_spec(_DEG_W), _full_spec((_D, _D))],
    out_specs=_row_spec(_D),
    out_shape=jax.ShapeDtypeStruct((_N_PAD, _D), jnp.float32),
)

_d_call = pl.pallas_call(
    _d_kernel,
    grid=(_N_BLOCKS,),
    in_specs=[_part_spec(_D), _row_spec(_D), _part_spec(_DEG_W),
              _full_spec((_D, _D)), _full_spec((1, _D))],
    out_specs=_row_spec(_D),
    out_shape=jax.ShapeDtypeStruct((_N_PAD, _D), jnp.float32),
)

_f_call = pl.pallas_call(
    _f_kernel,
    grid=(_N_BLOCKS,),
    in_specs=[_part_spec(_D), _row_spec(_D), _part_spec(_DEG_W),
              pl.BlockSpec((1, 1, _BM), lambda i: (i, 0, 0)),
              _full_spec((1, _D)), _full_spec((_D, 64)), _full_spec((1, 64)),
              _full_spec((64, _D)), _full_spec((1, _D))],
    out_specs=_full_spec((_G, _D)),
    out_shape=jax.ShapeDtypeStruct((_G, _D), jnp.float32),
    scratch_shapes=[pltpu.VMEM((_G, _D), jnp.float32),
                    pltpu.VMEM((_G, _D), jnp.float32)],
)


@jax.jit
def kernel(x, edge_index, batch, W1, b1, W2, b2, Wf, bf, Wo, bo):
    padv = _N + jnp.arange(_E_PAD - _E, dtype=jnp.int32) % (_N_PAD - _N)
    src = jnp.concatenate([edge_index[0], padv])
    dst = jnp.concatenate([edge_index[1], padv])
    x_pad = jnp.pad(x, ((0, _N_PAD - _N), (0, 0)))
    batch_pad = jnp.pad(batch, (0, _N_PAD - _N), constant_values=_G)
    batch_pad = batch_pad.reshape(_N_BLOCKS, 1, _BM)

    onehot_rows = jnp.zeros((2, _CHUNK, _DEG_W), jnp.float32).at[0, :, 0].set(1.0)
    degp = _deg_kernel()(dst, onehot_rows)
    y1 = _b_call(x_pad, degp, W1)
    agg1 = _agg_kernel()(y1, src, dst)
    y2 = _d_call(agg1, y1, degp, W2, b1.reshape(1, _D))
    agg2 = _agg_kernel()(y2, src, dst)

    wo_pad = jnp.pad(Wo, ((0, 0), (0, _D - 1)))
    bo_pad = jnp.pad(bo, (0, _D - 1)).reshape(1, _D)
    out = _f_call(agg2, y2, degp, batch_pad, b2.reshape(1, _D),
                  Wf, bf.reshape(1, 64), wo_pad, bo_pad)
    return out[:, 0:1]


# speedup vs baseline: 2.3492x; 1.3211x over previous
import functools

import jax
import jax.numpy as jnp
from jax import lax
from jax.experimental import pallas as pl
from jax.experimental.pallas import tpu as pltpu
from jax.experimental.pallas import tpu_sc as plsc

_N = 10000
_E = 640000
_D = 128
_G = 16

_NC = 2
_NS = 16
_NW = _NC * _NS

_CHUNK = 128
_N_PAD = 10240
_ROWS_PER_TILE = _N_PAD // _NS
_N_CHUNKS = 157
_E_PER_W = _N_CHUNKS * _CHUNK
_E_PAD = _E_PER_W * _NW
_DEG_W = 32


def _sc_mesh():
    return plsc.VectorSubcoreMesh(
        core_axis_name="c", subcore_axis_name="s",
        num_cores=_NC, num_subcores=_NS,
    )


def _deg_body(dst_hbm, ones_hbm, out_hbm, ones_v, zeros_v, idx_v, acc_s):
    cid = lax.axis_index("c")
    sid = lax.axis_index("s")
    wid = sid * _NC + cid

    pltpu.sync_copy(ones_hbm.at[0], ones_v)
    pltpu.sync_copy(ones_hbm.at[1], zeros_v)

    def zero_blk(i, _):
        pltpu.sync_copy(
            zeros_v,
            acc_s.at[pl.ds(sid * _ROWS_PER_TILE + i * _CHUNK, _CHUNK)],
        )
        return 0

    lax.fori_loop(0, _ROWS_PER_TILE // _CHUNK, zero_blk, 0)
    plsc.subcore_barrier()

    base = wid * _E_PER_W

    def body(i, _):
        pltpu.sync_copy(dst_hbm.at[pl.ds(base + i * _CHUNK, _CHUNK)], idx_v)
        pltpu.sync_copy(ones_v, acc_s.at[idx_v], add=True)
        return 0

    lax.fori_loop(0, _N_CHUNKS, body, 0)
    plsc.subcore_barrier()

    pltpu.sync_copy(
        acc_s.at[pl.ds(sid * _ROWS_PER_TILE, _ROWS_PER_TILE)],
        out_hbm.at[cid, pl.ds(sid * _ROWS_PER_TILE, _ROWS_PER_TILE)],
    )


@functools.cache
def _deg_kernel():
    return pl.kernel(
        _deg_body,
        out_type=jax.ShapeDtypeStruct((_NC, _N_PAD, _DEG_W), jnp.float32),
        mesh=_sc_mesh(),
        scratch_types=[
            pltpu.VMEM((_CHUNK, _DEG_W), jnp.float32),
            pltpu.VMEM((_CHUNK, _DEG_W), jnp.float32),
            pltpu.VMEM((_CHUNK,), jnp.int32),
            pltpu.VMEM_SHARED((_N_PAD, _DEG_W), jnp.float32),
        ],
    )


def _agg_body(y_hbm, src_hbm, dst_hbm, out_hbm, sidx_v, didx_v, sidx2_v, didx2_v,
              rows0_v, acc_s, sem0, isem, isem2):
    cid = lax.axis_index("c")
    sid = lax.axis_index("s")
    wid = sid * _NC + cid

    def init_blk(i, _):
        r0 = sid * _ROWS_PER_TILE + i * _CHUNK
        pltpu.sync_copy(y_hbm.at[pl.ds(r0, _CHUNK)], acc_s.at[pl.ds(r0, _CHUNK)])
        return 0

    lax.fori_loop(0, _ROWS_PER_TILE // _CHUNK, init_blk, 0)
    plsc.subcore_barrier()

    base = wid * _E_PER_W

    pltpu.async_copy(src_hbm.at[pl.ds(base, _CHUNK)], sidx_v, isem)
    pltpu.async_copy(dst_hbm.at[pl.ds(base, _CHUNK)], didx_v, isem)

    def body(t, _):
        g = 2 * t
        off = base + g * _CHUNK

        @pl.when(g + 1 < _N_CHUNKS)
        def _():
            pltpu.async_copy(src_hbm.at[pl.ds(off + _CHUNK, _CHUNK)], sidx2_v, isem2)
            pltpu.async_copy(dst_hbm.at[pl.ds(off + _CHUNK, _CHUNK)], didx2_v, isem2)

        pltpu.make_async_copy(src_hbm.at[pl.ds(off, _CHUNK)], sidx_v, isem).wait()
        pltpu.make_async_copy(dst_hbm.at[pl.ds(off, _CHUNK)], didx_v, isem).wait()
        pltpu.async_copy(y_hbm.at[sidx_v], rows0_v, sem0).wait()
        pltpu.sync_copy(rows0_v, acc_s.at[didx_v], add=True)

        @pl.when(g + 1 < _N_CHUNKS)
        def _():
            @pl.when(g + 2 < _N_CHUNKS)
            def _():
                pltpu.async_copy(
                    src_hbm.at[pl.ds(off + 2 * _CHUNK, _CHUNK)], sidx_v, isem)
                pltpu.async_copy(
                    dst_hbm.at[pl.ds(off + 2 * _CHUNK, _CHUNK)], didx_v, isem)

            pltpu.make_async_copy(
                src_hbm.at[pl.ds(off + _CHUNK, _CHUNK)], sidx2_v, isem2).wait()
            pltpu.make_async_copy(
                dst_hbm.at[pl.ds(off + _CHUNK, _CHUNK)], didx2_v, isem2).wait()
            pltpu.async_copy(y_hbm.at[sidx2_v], rows0_v, sem0).wait()
            pltpu.sync_copy(rows0_v, acc_s.at[didx2_v], add=True)

        return 0

    lax.fori_loop(0, (_N_CHUNKS + 1) // 2, body, 0)
    plsc.subcore_barrier()

    def out_blk(i, _):
        r0 = sid * _ROWS_PER_TILE + i * _CHUNK
        pltpu.sync_copy(acc_s.at[pl.ds(r0, _CHUNK)], out_hbm.at[cid, pl.ds(r0, _CHUNK)])
        return 0

    lax.fori_loop(0, _ROWS_PER_TILE // _CHUNK, out_blk, 0)


@functools.cache
def _agg_kernel():
    return pl.kernel(
        _agg_body,
        out_type=jax.ShapeDtypeStruct((_NC, _N_PAD, _D), jnp.float32),
        mesh=_sc_mesh(),
        scratch_types=[
            pltpu.VMEM((_CHUNK,), jnp.int32),
            pltpu.VMEM((_CHUNK,), jnp.int32),
            pltpu.VMEM((_CHUNK,), jnp.int32),
            pltpu.VMEM((_CHUNK,), jnp.int32),
            pltpu.VMEM((_CHUNK, _D), jnp.float32),
            pltpu.VMEM_SHARED((_N_PAD, _D), jnp.float32),
            pltpu.SemaphoreType.DMA,
            pltpu.SemaphoreType.DMA,
            pltpu.SemaphoreType.DMA,
        ],
    )


_BM = 512
_N_BLOCKS = _N_PAD // _BM


def _dis_block(degp):
    deg = degp[0, :, 0:1] + degp[1, :, 0:1] + 1.0
    return lax.rsqrt(deg)


def _b_kernel(x_ref, degp_ref, w1_ref, y1_ref):
    dis = _dis_block(degp_ref[...])
    y1_ref[...] = (x_ref[...] @ w1_ref[...]) * dis


def _d_kernel(aggp_ref, y1_ref, degp_ref, w2_ref, b1_ref, y2_ref):
    dis = _dis_block(degp_ref[...])
    agg = aggp_ref[0] + aggp_ref[1] - y1_ref[...]
    h1 = jnp.maximum(agg * dis + b1_ref[...], 0.0)
    y2_ref[...] = (h1 @ w2_ref[...]) * dis


def _f_kernel(aggp_ref, y2_ref, degp_ref, batch_ref, b2_ref, wf_ref, bf_ref,
              wo_ref, bo_ref, out_ref, sum_s, cnt_s):
    i = pl.program_id(0)
    dis = _dis_block(degp_ref[...])
    agg = aggp_ref[0] + aggp_ref[1] - y2_ref[...]
    h2 = agg * dis + b2_ref[...]

    b = batch_ref[0, 0, :].reshape(_BM, 1)
    gid = lax.broadcasted_iota(jnp.int32, (_BM, _G), 1)
    p = jnp.where(b == gid, 1.0, 0.0)

    contract = (((0,), (0,)), ((), ()))
    psum = lax.dot_general(p, h2, contract,
                           precision=lax.Precision.HIGHEST)
    pcnt = lax.dot_general(p, jnp.ones((_BM, _D), jnp.float32), contract)

    @pl.when(i == 0)
    def _():
        sum_s[...] = jnp.zeros_like(sum_s)
        cnt_s[...] = jnp.zeros_like(cnt_s)

    sum_s[...] += psum
    cnt_s[...] += pcnt

    @pl.when(i == _N_BLOCKS - 1)
    def _():
        pooled = sum_s[...] / jnp.maximum(cnt_s[...], 1.0)
        hf = jnp.maximum(pooled @ wf_ref[...] + bf_ref[...], 0.0)
        out_ref[...] = hf @ wo_ref[...] + bo_ref[...]


def _row_spec(width):
    return pl.BlockSpec((_BM, width), lambda i: (i, 0))


def _part_spec(width):
    return pl.BlockSpec((_NC, _BM, width), lambda i: (0, i, 0))


def _full_spec(shape):
    return pl.BlockSpec(shape, lambda i: tuple(0 for _ in shape))


_b_call = pl.pallas_call(
    _b_kernel,
    grid=(_N_BLOCKS,),
    in_specs=[_row_spec(_D), _part_spec(_DEG_W), _full_spec((_D, _D))],
    out_specs=_row_spec(_D),
    out_shape=jax.ShapeDtypeStruct((_N_PAD, _D), jnp.float32),
)

_d_call = pl.pallas_call(
    _d_kernel,
    grid=(_N_BLOCKS,),
    in_specs=[_part_spec(_D), _row_spec(_D), _part_spec(_DEG_W),
              _full_spec((_D, _D)), _full_spec((1, _D))],
    out_specs=_row_spec(_D),
    out_shape=jax.ShapeDtypeStruct((_N_PAD, _D), jnp.float32),
)

_f_call = pl.pallas_call(
    _f_kernel,
    grid=(_N_BLOCKS,),
    in_specs=[_part_spec(_D), _row_spec(_D), _part_spec(_DEG_W),
              pl.BlockSpec((1, 1, _BM), lambda i: (i, 0, 0)),
              _full_spec((1, _D)), _full_spec((_D, 64)), _full_spec((1, 64)),
              _full_spec((64, _D)), _full_spec((1, _D))],
    out_specs=_full_spec((_G, _D)),
    out_shape=jax.ShapeDtypeStruct((_G, _D), jnp.float32),
    scratch_shapes=[pltpu.VMEM((_G, _D), jnp.float32),
                    pltpu.VMEM((_G, _D), jnp.float32)],
)


@jax.jit
def kernel(x, edge_index, batch, W1, b1, W2, b2, Wf, bf, Wo, bo):
    padv = _N + jnp.arange(_E_PAD - _E, dtype=jnp.int32) % (_N_PAD - _N)
    src = jnp.concatenate([edge_index[0], padv])
    dst = jnp.concatenate([edge_index[1], padv])
    x_pad = jnp.pad(x, ((0, _N_PAD - _N), (0, 0)))
    batch_pad = jnp.pad(batch, (0, _N_PAD - _N), constant_values=_G)
    batch_pad = batch_pad.reshape(_N_BLOCKS, 1, _BM)

    onehot_rows = jnp.zeros((2, _CHUNK, _DEG_W), jnp.float32).at[0, :, 0].set(1.0)
    degp = _deg_kernel()(dst, onehot_rows)
    y1 = _b_call(x_pad, degp, W1)
    agg1 = _agg_kernel()(y1, src, dst)
    y2 = _d_call(agg1, y1, degp, W2, b1.reshape(1, _D))
    agg2 = _agg_kernel()(y2, src, dst)

    wo_pad = jnp.pad(Wo, ((0, 0), (0, _D - 1)))
    bo_pad = jnp.pad(bo, (0, _D - 1)).reshape(1, _D)
    out = _f_call(agg2, y2, degp, batch_pad, b2.reshape(1, _D),
                  Wf, bf.reshape(1, 64), wo_pad, bo_pad)
    return out[:, 0:1]
